# baseline (speedup 1.0000x reference)
import functools

import jax
import jax.numpy as jnp
from jax import lax
from jax.experimental import pallas as pl
from jax.experimental.pallas import tpu as pltpu
from jax.experimental.pallas import tpu_sc as plsc

D = 1024
L = 16
NC, NS = 2, 16
NW = NC * NS
C = 16
NBIN = 4
NBOUT = 2


def _rsqrt_vec(x):
    i = lax.bitcast_convert_type(x, jnp.int32)
    i = jnp.int32(0x5F3759DF) - (i >> 1)
    y = lax.bitcast_convert_type(i, jnp.float32)
    for _ in range(3):
        y = y * (1.5 - 0.5 * x * y * y)
    return y


def _make_sc_kernel(B):
    rows_per_w = B // NW
    nchunk = rows_per_w // C
    nquad = nchunk // NBIN
    mesh = plsc.VectorSubcoreMesh(core_axis_name="c", subcore_axis_name="s")

    @functools.partial(
        pl.kernel,
        mesh=mesh,
        out_type=jax.ShapeDtypeStruct((B, D), jnp.float32),
        scratch_types=(
            [pltpu.VMEM((rows_per_w,), jnp.int32)]
            + [pltpu.VMEM((C, D), jnp.float32) for _ in range(NBIN + NBOUT)]
            + [pltpu.VMEM((L, L), jnp.float32),
               pltpu.VMEM((L,), jnp.float32)]
            + [pltpu.SemaphoreType.DMA for _ in range(NBIN + NBOUT)]
        ),
    )
    def k(pos_hbm, table_hbm, out_hbm, idx_v,
          bin0, bin1, bin2, bin3, bout0, bout1, accbuf, invbuf,
          gs0, gs1, gs2, gs3, ss0, ss1):
        wid = lax.axis_index("s") * NC + lax.axis_index("c")
        row0 = wid * rows_per_w
        pltpu.sync_copy(pos_hbm.at[pl.ds(row0, rows_per_w)], idx_v)

        bins = (bin0, bin1, bin2, bin3)
        bouts = (bout0, bout1)
        gsems = (gs0, gs1, gs2, gs3)
        ssems = (ss0, ss1)
        lane = lax.iota(jnp.int32, L)

        def gather_start(c, b):
            pltpu.async_copy(
                table_hbm.at[idx_v.at[pl.ds(c * C, C)]], bins[b], gsems[b])

        def gather_wait(b):
            pltpu.make_async_copy(
                table_hbm.at[idx_v.at[pl.ds(0, C)]], bins[b], gsems[b]).wait()

        def scatter_start(c, ob):
            pltpu.async_copy(
                bouts[ob], out_hbm.at[pl.ds(row0 + c * C, C)], ssems[ob])

        def scatter_wait(ob):
            pltpu.make_async_copy(
                bouts[ob], out_hbm.at[pl.ds(row0, C)], ssems[ob]).wait()

        def sumsq_row(src, r):
            accs = [jnp.zeros((L,), jnp.float32) for _ in range(8)]
            for j in range(D // L):
                v = src[r, pl.ds(j * L, L)]
                accs[j % 8] = accs[j % 8] + v * v
            acc01 = accs[0] + accs[1]
            acc23 = accs[2] + accs[3]
            acc45 = accs[4] + accs[5]
            acc67 = accs[6] + accs[7]
            return (acc01 + acc23) + (acc45 + acc67)

        def merge_to_inv():
            vecs = [accbuf[r, :] for r in range(L)]
            for o in (1, 2, 4, 8):
                perm = jnp.bitwise_xor(lane, o)
                pick_b = (jnp.bitwise_and(lane, o) != 0)
                nxt = []
                for i in range(0, len(vecs), 2):
                    a, bb = vecs[i], vecs[i + 1]
                    asum = a + a.at[perm].get(mode="promise_in_bounds")
                    bsum = bb + bb.at[perm].get(mode="promise_in_bounds")
                    nxt.append(jnp.where(pick_b, bsum, asum))
                vecs = nxt
            invbuf[:] = _rsqrt_vec(jnp.maximum(vecs[0], 1e-24))

        def fused_step(src, nxt, dst):
            inv_vec = invbuf[:]

            @plsc.parallel_loop(0, C)
            def row_body(r):
                inv = inv_vec.at[jnp.full((L,), r, jnp.int32)].get(
                    mode="promise_in_bounds")
                accs = [jnp.zeros((L,), jnp.float32) for _ in range(4)]
                for j in range(D // L):
                    dst[r, pl.ds(j * L, L)] = src[r, pl.ds(j * L, L)] * inv
                    v = nxt[r, pl.ds(j * L, L)]
                    accs[j % 4] = accs[j % 4] + v * v
                acc01 = accs[0] + accs[1]
                acc23 = accs[2] + accs[3]
                accbuf[r, :] = acc01 + acc23

            merge_to_inv()

        def scale_step(src, dst):
            inv_vec = invbuf[:]

            @plsc.parallel_loop(0, C)
            def row_body(r):
                inv = inv_vec.at[jnp.full((L,), r, jnp.int32)].get(
                    mode="promise_in_bounds")
                for j in range(D // L):
                    dst[r, pl.ds(j * L, L)] = src[r, pl.ds(j * L, L)] * inv

        for b in range(NBIN):
            gather_start(b, b)
        gather_wait(0)

        @plsc.parallel_loop(0, C)
        def pro_body(r):
            accbuf[r, :] = sumsq_row(bins[0], r)

        merge_to_inv()

        def quad_body(q, carry):
            for u in range(NBIN):
                c = q * NBIN + u
                ib, ob = u, u % NBOUT

                @pl.when(c >= NBOUT)
                def _():
                    scatter_wait(ob)

                if u < NBIN - 1:
                    gather_wait(ib + 1)
                    fused_step(bins[ib], bins[ib + 1], bouts[ob])
                else:
                    @pl.when(q < nquad - 1)
                    def _():
                        gather_wait(0)
                        fused_step(bins[ib], bins[0], bouts[ob])

                    @pl.when(q == nquad - 1)
                    def _():
                        scale_step(bins[ib], bouts[ob])

                scatter_start(c, ob)

                @pl.when(c + NBIN < nchunk)
                def _():
                    gather_start(c + NBIN, ib)

            return carry

        lax.fori_loop(0, nquad, quad_body, 0)
        scatter_wait(0)
        scatter_wait(1)

    return k


def kernel(positions, positional_embeddings):
    B = positions.size
    pos_flat = positions.reshape(-1).astype(jnp.int32)
    table = positional_embeddings.astype(jnp.float32)
    out = _make_sc_kernel(B)(pos_flat, table)
    return out.reshape(positions.shape + (D,))


# speedup vs baseline: 1.1762x; 1.1762x over previous
import functools

import jax
import jax.numpy as jnp
from jax import lax
from jax.experimental import pallas as pl
from jax.experimental.pallas import tpu as pltpu
from jax.experimental.pallas import tpu_sc as plsc

D = 1024
L = 16
NC, NS = 2, 16
NW = NC * NS
C = 16
NBIN = 4
NBOUT = 2


def _rsqrt_vec(x):
    i = lax.bitcast_convert_type(x, jnp.int32)
    i = jnp.int32(0x5F3759DF) - (i >> 1)
    y = lax.bitcast_convert_type(i, jnp.float32)
    for _ in range(3):
        y = y * (1.5 - 0.5 * x * y * y)
    return y


def _make_sc_kernel(B):
    rows_per_w = B // NW
    nchunk = rows_per_w // C
    nquad = nchunk // NBIN
    mesh = plsc.VectorSubcoreMesh(core_axis_name="c", subcore_axis_name="s")

    @functools.partial(
        pl.kernel,
        mesh=mesh,
        out_type=jax.ShapeDtypeStruct((B, D), jnp.float32),
        scratch_types=(
            [pltpu.VMEM((rows_per_w,), jnp.int32)]
            + [pltpu.VMEM((C, D), jnp.float32) for _ in range(NBIN + NBOUT)]
            + [pltpu.VMEM((L, L), jnp.float32),
               pltpu.VMEM((L,), jnp.float32)]
            + [pltpu.SemaphoreType.DMA for _ in range(NBIN + NBOUT)]
        ),
    )
    def k(pos_hbm, table_hbm, out_hbm, idx_v,
          bin0, bin1, bin2, bin3, bout0, bout1, accbuf, invbuf,
          gs0, gs1, gs2, gs3, ss0, ss1):
        wid = lax.axis_index("s") * NC + lax.axis_index("c")
        row0 = wid * rows_per_w
        pltpu.sync_copy(pos_hbm.at[pl.ds(row0, rows_per_w)], idx_v)

        bins = (bin0, bin1, bin2, bin3)
        bouts = (bout0, bout1)
        gsems = (gs0, gs1, gs2, gs3)
        ssems = (ss0, ss1)
        lane = lax.iota(jnp.int32, L)

        def gather_start(c, b):
            pltpu.async_copy(
                table_hbm.at[idx_v.at[pl.ds(c * C, C)]], bins[b], gsems[b])

        def gather_wait(b):
            pltpu.make_async_copy(
                table_hbm.at[idx_v.at[pl.ds(0, C)]], bins[b], gsems[b]).wait()

        def scatter_start(c, ob):
            pltpu.async_copy(
                bouts[ob], out_hbm.at[pl.ds(row0 + c * C, C)], ssems[ob])

        def scatter_wait(ob):
            pltpu.make_async_copy(
                bouts[ob], out_hbm.at[pl.ds(row0, C)], ssems[ob]).wait()

        def sumsq_row(src, r):
            accs = [jnp.zeros((L,), jnp.float32) for _ in range(8)]
            for j in range(D // L):
                v = src[r, pl.ds(j * L, L)]
                accs[j % 8] = accs[j % 8] + v * v
            acc01 = accs[0] + accs[1]
            acc23 = accs[2] + accs[3]
            acc45 = accs[4] + accs[5]
            acc67 = accs[6] + accs[7]
            return (acc01 + acc23) + (acc45 + acc67)

        def merge_to_inv():
            vecs = [accbuf[r, :] for r in range(L)]
            for o in (1, 2, 4, 8):
                perm = jnp.bitwise_xor(lane, o)
                pick_b = (jnp.bitwise_and(lane, o) != 0)
                nxt = []
                for i in range(0, len(vecs), 2):
                    a, bb = vecs[i], vecs[i + 1]
                    asum = a + a.at[perm].get(mode="promise_in_bounds")
                    bsum = bb + bb.at[perm].get(mode="promise_in_bounds")
                    nxt.append(jnp.where(pick_b, bsum, asum))
                vecs = nxt
            invbuf[:] = _rsqrt_vec(jnp.maximum(vecs[0], 1e-24))

        def fused_step(src, nxt, dst):
            inv_vec = invbuf[:]

            def row_body(r, iv):
                inv = iv.at[jnp.full((L,), r, jnp.int32)].get(
                    mode="promise_in_bounds")
                accs = [jnp.zeros((L,), jnp.float32) for _ in range(4)]
                for j in range(D // L):
                    dst[r, pl.ds(j * L, L)] = src[r, pl.ds(j * L, L)] * inv
                    v = nxt[r, pl.ds(j * L, L)]
                    accs[j % 4] = accs[j % 4] + v * v
                acc01 = accs[0] + accs[1]
                acc23 = accs[2] + accs[3]
                accbuf[r, :] = acc01 + acc23
                return iv

            lax.fori_loop(0, C, row_body, inv_vec)
            merge_to_inv()

        def scale_step(src, dst):
            inv_vec = invbuf[:]

            def row_body(r, iv):
                inv = iv.at[jnp.full((L,), r, jnp.int32)].get(
                    mode="promise_in_bounds")
                for j in range(D // L):
                    dst[r, pl.ds(j * L, L)] = src[r, pl.ds(j * L, L)] * inv
                return iv

            lax.fori_loop(0, C, row_body, inv_vec)

        for b in range(NBIN):
            gather_start(b, b)
        gather_wait(0)

        def pro_body(r, carry):
            accbuf[r, :] = sumsq_row(bins[0], r)
            return carry

        lax.fori_loop(0, C, pro_body, 0)
        merge_to_inv()

        def quad_body(q, carry):
            for u in range(NBIN):
                c = q * NBIN + u
                ib, ob = u, u % NBOUT

                @pl.when(c >= NBOUT)
                def _():
                    scatter_wait(ob)

                if u < NBIN - 1:
                    gather_wait(ib + 1)
                    fused_step(bins[ib], bins[ib + 1], bouts[ob])
                else:
                    @pl.when(q < nquad - 1)
                    def _():
                        gather_wait(0)
                        fused_step(bins[ib], bins[0], bouts[ob])

                    @pl.when(q == nquad - 1)
                    def _():
                        scale_step(bins[ib], bouts[ob])

                scatter_start(c, ob)

                @pl.when(c + NBIN < nchunk)
                def _():
                    gather_start(c + NBIN, ib)

            return carry

        lax.fori_loop(0, nquad, quad_body, 0)
        scatter_wait(0)
        scatter_wait(1)

    return k


def kernel(positions, positional_embeddings):
    B = positions.size
    pos_flat = positions.reshape(-1).astype(jnp.int32)
    table = positional_embeddings.astype(jnp.float32)
    out = _make_sc_kernel(B)(pos_flat, table)
    return out.reshape(positions.shape + (D,))


# speedup vs baseline: 1.1859x; 1.0082x over previous
import functools

import jax
import jax.numpy as jnp
from jax import lax
from jax.experimental import pallas as pl
from jax.experimental.pallas import tpu as pltpu
from jax.experimental.pallas import tpu_sc as plsc

D = 1024
L = 16
NC, NS = 2, 16
NW = NC * NS
C = 16
NBIN = 4
NBOUT = 2


def _rsqrt_vec(x):
    i = lax.bitcast_convert_type(x, jnp.int32)
    i = jnp.int32(0x5F3759DF) - (i >> 1)
    y = lax.bitcast_convert_type(i, jnp.float32)
    for _ in range(3):
        y = y * (1.5 - 0.5 * x * y * y)
    return y


def _make_sc_kernel(B):
    rows_per_w = B // NW
    nchunk = rows_per_w // C
    nquad = nchunk // NBIN
    mesh = plsc.VectorSubcoreMesh(core_axis_name="c", subcore_axis_name="s")

    @functools.partial(
        pl.kernel,
        mesh=mesh,
        out_type=jax.ShapeDtypeStruct((B, D), jnp.float32),
        scratch_types=(
            [pltpu.VMEM((rows_per_w,), jnp.int32)]
            + [pltpu.VMEM((C, D), jnp.float32) for _ in range(NBIN + NBOUT)]
            + [pltpu.VMEM((L, L), jnp.float32),
               pltpu.VMEM((L,), jnp.float32)]
            + [pltpu.SemaphoreType.DMA for _ in range(NBIN + NBOUT)]
        ),
    )
    def k(pos_hbm, table_hbm, out_hbm, idx_v,
          bin0, bin1, bin2, bin3, bout0, bout1, accbuf, invbuf,
          gs0, gs1, gs2, gs3, ss0, ss1):
        wid = lax.axis_index("s") * NC + lax.axis_index("c")
        row0 = wid * rows_per_w
        pltpu.sync_copy(pos_hbm.at[pl.ds(row0, rows_per_w)], idx_v)

        bins = (bin0, bin1, bin2, bin3)
        bouts = (bout0, bout1)
        gsems = (gs0, gs1, gs2, gs3)
        ssems = (ss0, ss1)
        lane = lax.iota(jnp.int32, L)

        def gather_start(c, b):
            pltpu.async_copy(
                table_hbm.at[idx_v.at[pl.ds(c * C, C)]], bins[b], gsems[b])

        def gather_wait(b):
            pltpu.make_async_copy(
                table_hbm.at[idx_v.at[pl.ds(0, C)]], bins[b], gsems[b]).wait()

        def scatter_start(c, ob):
            pltpu.async_copy(
                bouts[ob], out_hbm.at[pl.ds(row0 + c * C, C)], ssems[ob])

        def scatter_wait(ob):
            pltpu.make_async_copy(
                bouts[ob], out_hbm.at[pl.ds(row0, C)], ssems[ob]).wait()

        def sumsq_row(src, r):
            accs = [jnp.zeros((L,), jnp.float32) for _ in range(8)]
            for j in range(D // L):
                v = src[r, pl.ds(j * L, L)]
                accs[j % 8] = accs[j % 8] + v * v
            acc01 = accs[0] + accs[1]
            acc23 = accs[2] + accs[3]
            acc45 = accs[4] + accs[5]
            acc67 = accs[6] + accs[7]
            return (acc01 + acc23) + (acc45 + acc67)

        def merge_to_inv():
            vecs = [accbuf[r, :] for r in range(L)]
            for o in (1, 2, 4, 8):
                perm = jnp.bitwise_xor(lane, o)
                pick_b = (jnp.bitwise_and(lane, o) != 0)
                nxt = []
                for i in range(0, len(vecs), 2):
                    a, bb = vecs[i], vecs[i + 1]
                    asum = a + a.at[perm].get(mode="promise_in_bounds")
                    bsum = bb + bb.at[perm].get(mode="promise_in_bounds")
                    nxt.append(jnp.where(pick_b, bsum, asum))
                vecs = nxt
            invbuf[:] = _rsqrt_vec(jnp.maximum(vecs[0], 1e-24))

        def fused_step(src, nxt, dst):
            inv_vec = invbuf[:]

            def row_body(r, iv):
                inv = iv.at[jnp.full((L,), r, jnp.int32)].get(
                    mode="promise_in_bounds")
                accs = [jnp.zeros((L,), jnp.float32) for _ in range(2)]
                for j in range(D // L):
                    dst[r, pl.ds(j * L, L)] = src[r, pl.ds(j * L, L)] * inv
                    v = nxt[r, pl.ds(j * L, L)]
                    accs[j % 2] = accs[j % 2] + v * v
                accbuf[r, :] = accs[0] + accs[1]
                return iv

            lax.fori_loop(0, C, row_body, inv_vec)
            merge_to_inv()

        def scale_step(src, dst):
            inv_vec = invbuf[:]

            def row_body(r, iv):
                inv = iv.at[jnp.full((L,), r, jnp.int32)].get(
                    mode="promise_in_bounds")
                for j in range(D // L):
                    dst[r, pl.ds(j * L, L)] = src[r, pl.ds(j * L, L)] * inv
                return iv

            lax.fori_loop(0, C, row_body, inv_vec)

        for b in range(NBIN):
            gather_start(b, b)
        gather_wait(0)

        def pro_body(r, carry):
            accbuf[r, :] = sumsq_row(bins[0], r)
            return carry

        lax.fori_loop(0, C, pro_body, 0)
        merge_to_inv()

        def quad_body(q, carry):
            for u in range(NBIN):
                c = q * NBIN + u
                ib, ob = u, u % NBOUT

                @pl.when(c >= NBOUT)
                def _():
                    scatter_wait(ob)

                if u < NBIN - 1:
                    gather_wait(ib + 1)
                    fused_step(bins[ib], bins[ib + 1], bouts[ob])
                else:
                    @pl.when(q < nquad - 1)
                    def _():
                        gather_wait(0)
                        fused_step(bins[ib], bins[0], bouts[ob])

                    @pl.when(q == nquad - 1)
                    def _():
                        scale_step(bins[ib], bouts[ob])

                scatter_start(c, ob)

                @pl.when(c + NBIN < nchunk)
                def _():
                    gather_start(c + NBIN, ib)

            return carry

        lax.fori_loop(0, nquad, quad_body, 0)
        scatter_wait(0)
        scatter_wait(1)

    return k


def kernel(positions, positional_embeddings):
    B = positions.size
    pos_flat = positions.reshape(-1).astype(jnp.int32)
    table = positional_embeddings.astype(jnp.float32)
    out = _make_sc_kernel(B)(pos_flat, table)
    return out.reshape(positions.shape + (D,))
